# Initial kernel scaffold; baseline (speedup 1.0000x reference)
import jax
import jax.numpy as jnp
from jax.experimental import pallas as pl


def kernel(grid, color_table, row_table, col_table):
    raise NotImplementedError("write your pallas kernel here")


# speedup vs baseline: 3.0010x; 3.0010x over previous
import functools

import jax
import jax.numpy as jnp
from jax import lax
from jax.experimental import pallas as pl
from jax.experimental.pallas import tpu as pltpu
from jax.experimental.pallas import tpu_sc as plsc

D_MODEL = 128
H = 30
W = 30
NCOLORS = 11
P = H * W
B = 1024
TOTAL = B * P
NC, NS = 2, 16
NW = NC * NS
RPW = TOTAL // NW
CHUNK = 128
NCH = RPW // CHUNK


def _fused_body(color_ref, row_ref, col_ref, out_ref):
    out_ref[...] = (color_ref[...][:, None, None, :]
                    + row_ref[...][None, :, None, :]
                    + col_ref[...][None, None, :, :])


def _build_fused(color_table, row_table, col_table):
    out = pl.pallas_call(
        _fused_body,
        out_shape=jax.ShapeDtypeStruct((NCOLORS, H, W, D_MODEL), jnp.float32),
    )(color_table, row_table, col_table)
    return out.reshape(NCOLORS * P, D_MODEL)


_mesh = plsc.VectorSubcoreMesh(core_axis_name="c", subcore_axis_name="s",
                               num_cores=NC, num_subcores=NS)


@functools.partial(
    pl.kernel,
    out_type=jax.ShapeDtypeStruct((TOTAL, D_MODEL), jnp.float32),
    mesh=_mesh,
    scratch_types=[
        pltpu.VMEM((NCH, CHUNK), jnp.int32),
        pltpu.VMEM((2, CHUNK, D_MODEL), jnp.float32),
        pltpu.SemaphoreType.DMA,
        pltpu.SemaphoreType.DMA,
    ],
)
def _sc_gather(fused_hbm, grid_hbm, out_hbm, idx_v, rows_v, gsem, ssem):
    wid = lax.axis_index("s") * NC + lax.axis_index("c")
    base = wid * RPW

    pltpu.sync_copy(grid_hbm.at[wid], idx_v)

    def idx_body(j, p):
        for i in range(CHUNK // 16):
            sl = pl.ds(i * 16, 16)
            idx_v[j, sl] = idx_v[j, sl] * P + p
            p = p + 16
            p = jnp.where(p >= P, p - P, p)
        return p

    lax.fori_loop(0, NCH, idx_body, lax.iota(jnp.int32, 16))

    def ch_body(j, _):
        pltpu.async_copy(fused_hbm.at[idx_v.at[j]], rows_v.at[0], gsem).wait()
        pltpu.sync_copy(rows_v.at[0],
                        out_hbm.at[pl.ds(base + j * CHUNK, CHUNK)])
        return 0

    lax.fori_loop(0, NCH, ch_body, 0)


def kernel(grid, color_table, row_table, col_table):
    fused = _build_fused(color_table, row_table, col_table)
    grid3 = grid.reshape(NW, NCH, CHUNK)
    out = _sc_gather(fused, grid3)
    return out.reshape(B, H, W, D_MODEL)


# speedup vs baseline: 3.2071x; 1.0687x over previous
import functools

import jax
import jax.numpy as jnp
from jax import lax
from jax.experimental import pallas as pl
from jax.experimental.pallas import tpu as pltpu
from jax.experimental.pallas import tpu_sc as plsc

D_MODEL = 128
H = 30
W = 30
NCOLORS = 11
P = H * W
B = 1024
TOTAL = B * P
NC, NS = 2, 16
NW = NC * NS
RPW = TOTAL // NW
CHUNK = 128
NCH = RPW // CHUNK


def _fused_body(color_ref, row_ref, col_ref, out_ref):
    out_ref[...] = (color_ref[...][:, None, None, :]
                    + row_ref[...][None, :, None, :]
                    + col_ref[...][None, None, :, :])


def _build_fused(color_table, row_table, col_table):
    out = pl.pallas_call(
        _fused_body,
        out_shape=jax.ShapeDtypeStruct((NCOLORS, H, W, D_MODEL), jnp.float32),
    )(color_table, row_table, col_table)
    return out.reshape(NCOLORS * P, D_MODEL)


_mesh = plsc.VectorSubcoreMesh(core_axis_name="c", subcore_axis_name="s",
                               num_cores=NC, num_subcores=NS)


NBUF = 3


@functools.partial(
    pl.kernel,
    out_type=jax.ShapeDtypeStruct((TOTAL, D_MODEL), jnp.float32),
    mesh=_mesh,
    scratch_types=[
        pltpu.VMEM((NCH, CHUNK), jnp.int32),
        pltpu.VMEM((NBUF, CHUNK, D_MODEL), jnp.float32),
        [pltpu.SemaphoreType.DMA] * NBUF,
        [pltpu.SemaphoreType.DMA] * NBUF,
    ],
)
def _sc_gather(fused_hbm, grid_hbm, out_hbm, idx_v, rows_v, gsems, ssems):
    wid = lax.axis_index("s") * NC + lax.axis_index("c")
    base = wid * RPW

    pltpu.sync_copy(grid_hbm.at[wid], idx_v)

    def idx_body(j, p):
        for i in range(CHUNK // 16):
            sl = pl.ds(i * 16, 16)
            idx_v[j, sl] = idx_v[j, sl] * P + p
            p = p + 16
            p = jnp.where(p >= P, p - P, p)
        return p

    lax.fori_loop(0, NCH, idx_body, lax.iota(jnp.int32, 16))

    def g_desc(j, b):
        return pltpu.make_async_copy(
            fused_hbm.at[idx_v.at[j]], rows_v.at[b], gsems[b])

    def s_desc(j, b):
        return pltpu.make_async_copy(
            rows_v.at[b], out_hbm.at[pl.ds(base + j * CHUNK, CHUNK)], ssems[b])

    def step(j, b, wait_prev_scatter):
        g_desc(j, b).wait()
        s_desc(j, b).start()
        nb = (b + 1) % NBUF
        if wait_prev_scatter:
            s_desc(j + 1 - NBUF, nb).wait()
        return nb

    g_desc(0, 0).start()
    for j in range(NBUF):
        nb = step(j, j % NBUF, wait_prev_scatter=(j == NBUF - 1))
        g_desc(j + 1, nb).start()

    def outer(t, _):
        for b in range(NBUF):
            j = t * NBUF + b
            step(j, b, wait_prev_scatter=True)
            g_desc(j + 1, (b + 1) % NBUF).start()
        return 0

    lax.fori_loop(1, NCH // NBUF - 1, outer, 0)

    for j in range(NCH - NBUF, NCH):
        b = j % NBUF
        g_desc(j, b).wait()
        s_desc(j, b).start()
        if j + 1 < NCH:
            nb = (b + 1) % NBUF
            s_desc(j + 1 - NBUF, nb).wait()
            g_desc(j + 1, nb).start()
    for j in range(NCH - NBUF, NCH):
        s_desc(j, j % NBUF).wait()


def kernel(grid, color_table, row_table, col_table):
    fused = _build_fused(color_table, row_table, col_table)
    grid3 = grid.reshape(NW, NCH, CHUNK)
    out = _sc_gather(fused, grid3)
    return out.reshape(B, H, W, D_MODEL)
